# baseline (speedup 1.0000x reference)
import functools

import jax
import jax.numpy as jnp
from jax.experimental import pallas as pl
from jax.experimental.pallas import tpu as pltpu


def _pick_block(n: int, target: int) -> int:
    best = 0
    for d in range(8, min(n, target) + 1, 8):
        if n % d == 0:
            best = d
    return best if best else n


def _gcn_body(x_ref, w1_ref, wc_ref, adj_t_ref, adj_b_ref, muvar_ref,
              h0_ref, h1c_ref, *, p, bm):
    i = pl.program_id(0)
    half = bm // 2
    row = jax.lax.rem(i, p) * bm

    @pl.when(i == 0)
    def _proj_x():
        h0_ref[...] = jnp.dot(x_ref[...], w1_ref[...],
                              preferred_element_type=jnp.float32)

    @pl.when(i < p)
    def _phase_a():
        for off, ref in ((0, adj_t_ref), (half, adj_b_ref)):
            acc = jnp.dot(ref[...], h0_ref[...],
                          preferred_element_type=jnp.float32)
            acc = jnp.maximum(acc, 0.0)
            h1c_ref[pl.ds(row + off, half), :] = jnp.dot(
                acc, wc_ref[...], preferred_element_type=jnp.float32)

    @pl.when(i >= p)
    def _phase_b():
        for off, ref in ((0, adj_t_ref), (half, adj_b_ref)):
            muvar_ref[pl.ds(off, half), :] = jnp.dot(
                ref[...], h1c_ref[...], preferred_element_type=jnp.float32)


def _gram_body(a_ref, b_ref, o_ref):
    o_ref[...] = jax.lax.dot_general(
        a_ref[...], b_ref[...],
        (((1,), (1,)), ((), ())),
        preferred_element_type=jnp.float32)


def kernel(x, adj, W1, W2, W3):
    n = adj.shape[0]
    nfeat = x.shape[1]
    nhid = W1.shape[1]
    bm = _pick_block(n, 400)
    p = n // bm
    bg = _pick_block(n, 400)
    pg = n // bg

    wc = jnp.concatenate([W2, W3], axis=1)

    muvar = pl.pallas_call(
        functools.partial(_gcn_body, p=p, bm=bm),
        grid=(2 * p,),
        in_specs=[
            pl.BlockSpec((n, nfeat), lambda i: (0, 0)),
            pl.BlockSpec((nfeat, nhid), lambda i: (0, 0)),
            pl.BlockSpec((nhid, 2 * nhid), lambda i: (0, 0)),
            pl.BlockSpec((bm // 2, n),
                         lambda i: (2 * jax.lax.rem(i, p), 0)),
            pl.BlockSpec((bm // 2, n),
                         lambda i: (2 * jax.lax.rem(i, p) + 1, 0)),
        ],
        out_specs=pl.BlockSpec((bm, 2 * nhid),
                               lambda i: (jnp.clip(i - p, 0, p - 1), 0)),
        out_shape=jax.ShapeDtypeStruct((n, 2 * nhid), jnp.float32),
        scratch_shapes=[
            pltpu.VMEM((n, nhid), jnp.float32),
            pltpu.VMEM((n, 2 * nhid), jnp.float32),
        ],
        compiler_params=pltpu.CompilerParams(
            dimension_semantics=("arbitrary",)),
    )(x, W1, wc, adj, adj)

    mu = muvar[:, :nhid]
    logvar = muvar[:, nhid:]

    recon = pl.pallas_call(
        _gram_body,
        grid=(pg,),
        in_specs=[
            pl.BlockSpec((bg, nhid), lambda i: (i, 0)),
            pl.BlockSpec((n, nhid), lambda i: (0, 0)),
        ],
        out_specs=pl.BlockSpec((bg, n), lambda i: (i, 0)),
        out_shape=jax.ShapeDtypeStruct((n, n), jnp.float32),
        compiler_params=pltpu.CompilerParams(
            dimension_semantics=("parallel",)),
    )(mu, mu)

    return (recon, mu, logvar)


# speedup vs baseline: 1.0365x; 1.0365x over previous
import functools

import jax
import jax.numpy as jnp
from jax.experimental import pallas as pl
from jax.experimental.pallas import tpu as pltpu


def _pick_block(n: int, target: int) -> int:
    best = 0
    for d in range(8, min(n, target) + 1, 8):
        if n % d == 0:
            best = d
    return best if best else n


def _gcn_body(x_ref, w1_ref, wc_ref, adj_ref, muvar_ref, h0_ref, h1c_ref,
              *, p, bm):
    i = pl.program_id(0)
    row = jax.lax.rem(i, p) * bm

    @pl.when(i == 0)
    def _proj_x():
        h0_ref[...] = jnp.dot(x_ref[...], w1_ref[...],
                              preferred_element_type=jnp.float32)

    @pl.when(i < p)
    def _phase_a():
        acc = jnp.dot(adj_ref[...], h0_ref[...],
                      preferred_element_type=jnp.float32)
        acc = jnp.maximum(acc, 0.0)
        h1c_ref[pl.ds(row, bm), :] = jnp.dot(
            acc, wc_ref[...], preferred_element_type=jnp.float32)

    @pl.when(i >= p)
    def _phase_b():
        muvar_ref[...] = jnp.dot(adj_ref[...], h1c_ref[...],
                                 preferred_element_type=jnp.float32)


def _gram_body(a_ref, b_ref, o_ref):
    o_ref[...] = jax.lax.dot_general(
        a_ref[...], b_ref[...],
        (((1,), (1,)), ((), ())),
        preferred_element_type=jnp.float32)


def kernel(x, adj, W1, W2, W3):
    n = adj.shape[0]
    nfeat = x.shape[1]
    nhid = W1.shape[1]
    bm = _pick_block(n, 400)
    p = n // bm
    bg = _pick_block(n, 400)
    pg = n // bg

    wc = jnp.concatenate([W2, W3], axis=1)

    muvar = pl.pallas_call(
        functools.partial(_gcn_body, p=p, bm=bm),
        grid=(2 * p,),
        in_specs=[
            pl.BlockSpec((n, nfeat), lambda i: (0, 0)),
            pl.BlockSpec((nfeat, nhid), lambda i: (0, 0)),
            pl.BlockSpec((nhid, 2 * nhid), lambda i: (0, 0)),
            pl.BlockSpec((bm, n), lambda i: (jax.lax.rem(i, p), 0)),
        ],
        out_specs=pl.BlockSpec((bm, 2 * nhid),
                               lambda i: (jnp.clip(i - p, 0, p - 1), 0)),
        out_shape=jax.ShapeDtypeStruct((n, 2 * nhid), jnp.float32),
        scratch_shapes=[
            pltpu.VMEM((n, nhid), jnp.float32),
            pltpu.VMEM((n, 2 * nhid), jnp.float32),
        ],
        compiler_params=pltpu.CompilerParams(
            dimension_semantics=("arbitrary",)),
    )(x, W1, wc, adj)

    mu = muvar[:, :nhid]
    logvar = muvar[:, nhid:]

    recon = pl.pallas_call(
        _gram_body,
        grid=(pg,),
        in_specs=[
            pl.BlockSpec((bg, nhid), lambda i: (i, 0)),
            pl.BlockSpec((n, nhid), lambda i: (0, 0)),
        ],
        out_specs=pl.BlockSpec((bg, n), lambda i: (i, 0)),
        out_shape=jax.ShapeDtypeStruct((n, n), jnp.float32),
        compiler_params=pltpu.CompilerParams(
            dimension_semantics=("arbitrary",)),
    )(mu, mu)

    return (recon, mu, logvar)
